# baseline (speedup 1.0000x reference)
import functools

import jax
import jax.numpy as jnp
from jax import lax
from jax.experimental import pallas as pl
from jax.experimental.pallas import tpu as pltpu
from jax.experimental.pallas import tpu_sc as plsc

E = 128
V = 128
F = 7
T = 15
SCALE = 1.0 / (E ** 0.5)
NW = 16
PW = V // NW
IW = 64


def _sc_compose(w_hbm, ft_hbm, out_hbm, idx_v, rows_v, stage_v, sem):
    wid = lax.axis_index("s") * 2 + lax.axis_index("c")

    @pl.when(wid < NW)
    def _work():
        pltpu.sync_copy(ft_hbm, idx_v)
        pltpu.async_copy(w_hbm.at[idx_v.at[wid]], rows_v, sem).wait()
        for p in range(PW):
            for l in range(E // 16):
                acc = rows_v[p * F, pl.ds(16 * l, 16)]
                for k in range(1, F):
                    acc = acc + rows_v[p * F + k, pl.ds(16 * l, 16)]
                stage_v[p, pl.ds(16 * l, 16)] = acc * SCALE
        pltpu.sync_copy(stage_v, out_hbm.at[pl.ds(wid * PW, PW)])


@functools.partial(
    pl.kernel,
    mesh=plsc.VectorSubcoreMesh(core_axis_name="c", subcore_axis_name="s"),
    out_type=jax.ShapeDtypeStruct((V, E), jnp.float32),
    scratch_types=[
        pltpu.VMEM((NW, IW), jnp.int32),
        pltpu.VMEM((IW, E), jnp.float32),
        pltpu.VMEM((PW, E), jnp.float32),
        pltpu.SemaphoreType.DMA,
    ],
)
def _compose_call(w_hbm, ft_hbm, out_hbm, idx_v, rows_v, stage_v, sem):
    _sc_compose(w_hbm, ft_hbm, out_hbm, idx_v, rows_v, stage_v, sem)


def _tc_body(cr_ref, w_ref, x_ref, o_ref, mw_ref):
    @pl.when(pl.program_id(0) == 0)
    def _assemble():
        mw_ref[0:1, :] = w_ref[0:1, :] * SCALE
        mw_ref[1:V + 1, :] = cr_ref[...]

    o_ref[...] = lax.dot_general(mw_ref[...], x_ref[...],
                                 (((1,), (1,)), ((), ())),
                                 preferred_element_type=jnp.float32)


@jax.jit
def kernel(inputs, weight, feature_table):
    B = inputs.shape[0]
    idx_mat = jnp.pad(feature_table.reshape(NW, PW * F), ((0, 0), (0, IW - PW * F)))
    cr = _compose_call(weight, idx_mat)
    blk = 8192
    grid = (B // blk,)
    ot = pl.pallas_call(
        _tc_body,
        grid=grid,
        in_specs=[
            pl.BlockSpec((V, E), lambda i: (0, 0)),
            pl.BlockSpec((T, E), lambda i: (0, 0)),
            pl.BlockSpec((blk, E), lambda i: (i, 0)),
        ],
        out_specs=pl.BlockSpec((V + 1, blk), lambda i: (0, i)),
        out_shape=jax.ShapeDtypeStruct((V + 1, B), jnp.float32),
        scratch_shapes=[pltpu.VMEM((V + 1, E), jnp.float32)],
    )(cr, weight, inputs)
    return ot.T


# speedup vs baseline: 3.9403x; 3.9403x over previous
import jax
import jax.numpy as jnp
from jax import lax
from jax.experimental import pallas as pl
from jax.experimental.pallas import tpu as pltpu

E = 128
V = 128
F = 7
T = 15
SCALE = 1.0 / (E ** 0.5)


def _body(x_ref, w_ref, ft_ref, o_ref, mw_ref):
    @pl.when(pl.program_id(0) == 0)
    def _compose():
        ft = ft_ref[...]
        t_row = lax.broadcasted_iota(jnp.int32, (V, T), 1)
        m = jnp.zeros((V, T), jnp.float32)
        for jj in range(F):
            m = m + (ft[:, jj:jj + 1] == t_row).astype(jnp.float32)
        row0 = (lax.broadcasted_iota(jnp.int32, (1, T), 1) == 0).astype(jnp.float32)
        m_full = jnp.concatenate([row0, m], axis=0)
        mw_ref[...] = lax.dot_general(m_full, w_ref[...],
                                      (((1,), (0,)), ((), ())),
                                      preferred_element_type=jnp.float32) * SCALE

    o_ref[...] = lax.dot_general(mw_ref[...], x_ref[...],
                                 (((1,), (1,)), ((), ())),
                                 preferred_element_type=jnp.float32)


@jax.jit
def kernel(inputs, weight, feature_table):
    B = inputs.shape[0]
    blk = 8192
    grid = (B // blk,)
    ot = pl.pallas_call(
        _body,
        grid=grid,
        in_specs=[
            pl.BlockSpec((blk, E), lambda i: (i, 0)),
            pl.BlockSpec((T, E), lambda i: (0, 0)),
            pl.BlockSpec((V, F), lambda i: (0, 0)),
        ],
        out_specs=pl.BlockSpec((V + 1, blk), lambda i: (0, i)),
        out_shape=jax.ShapeDtypeStruct((V + 1, B), jnp.float32),
        scratch_shapes=[pltpu.VMEM((V + 1, E), jnp.float32)],
    )(inputs, weight, feature_table)
    return ot.T
